# baseline (speedup 1.0000x reference)
import functools

import jax
import jax.numpy as jnp
from jax.experimental import pallas as pl
from jax.experimental.pallas import tpu as pltpu

N_CONST = 100000
D = 128
NUM_GRAPHS = 512
BLOCK = 8192
SEG_TILE = 32
N_TILES = NUM_GRAPHS // SEG_TILE


def _readout_kernel(first_ref, last_ref, b_ref, x_ref, w1_ref, b1_ref,
                    w2_ref, out_ref, m_scr, s_scr, v_scr, *, nb, n):
    i = pl.program_id(0)

    @pl.when(i == 0)
    def _init():
        m_scr[...] = jnp.zeros_like(m_scr)
        s_scr[...] = jnp.zeros_like(s_scr)
        v_scr[...] = jnp.zeros_like(v_scr)

    x_blk = x_ref[...]
    if n % BLOCK:
        @pl.when(i == nb - 1)
        def _mask_tail():
            row = jax.lax.broadcasted_iota(jnp.int32, (BLOCK, 1), 0)
            x_ref[...] = jnp.where(row < (n - (nb - 1) * BLOCK),
                                   x_blk, 0.0)
        x_blk = x_ref[...]
    x_bf = x_blk.astype(jnp.bfloat16)
    h = jnp.tanh(
        jax.lax.dot_general(x_bf, w1_ref[...], (((1,), (1,)), ((), ())),
                            preferred_element_type=jnp.float32)
        + b1_ref[...])
    sc = jax.lax.dot_general(w2_ref[...], h.astype(jnp.bfloat16),
                             (((1,), (1,)), ((), ())),
                             preferred_element_type=jnp.float32)

    m_old = m_scr[...]
    m_new = jnp.maximum(m_old, jnp.max(sc, axis=(0, 1), keepdims=True))
    m_scr[...] = m_new
    e = jnp.exp(sc - m_new).astype(jnp.bfloat16)

    @pl.when(m_new[0, 0] > m_old[0, 0])
    def _rescale():
        r = jnp.exp(m_old - m_new)[0, 0]
        s_scr[...] = s_scr[...] * r
        v_scr[...] = v_scr[...] * r

    ids = b_ref[0]
    first, last = first_ref[i], last_ref[i]
    for j in range(N_TILES):
        lo = j * SEG_TILE

        @pl.when(jnp.logical_and(first <= lo + SEG_TILE - 1, last >= lo))
        def _tile(lo=lo):
            seg_iota = lo + jax.lax.broadcasted_iota(
                jnp.int16, (SEG_TILE, 1), 0)
            zero = jnp.zeros((), jnp.bfloat16)
            ohw = jnp.where(seg_iota == ids, e, zero)
            sl = pl.ds(lo, SEG_TILE)
            ones = jnp.ones((BLOCK, 1), jnp.bfloat16)
            s_scr[sl, :] += jnp.dot(ohw, ones,
                                    preferred_element_type=jnp.float32)
            v_scr[sl, :] += jnp.dot(ohw, x_bf,
                                    preferred_element_type=jnp.float32)

    @pl.when(i == nb - 1)
    def _finish():
        s = s_scr[...]
        recip = jnp.where(s > 0.0, 1.0 / s, 0.0)
        out_ref[...] = v_scr[...] * recip


def kernel(x, batch, W1, b1, W2):
    n = x.shape[0]
    nb = (n + BLOCK - 1) // BLOCK
    npad = nb * BLOCK - n
    batch = batch.astype(jnp.int32)
    if npad:
        batch = jnp.pad(batch, (0, npad), constant_values=NUM_GRAPHS)
    firsts = batch[::BLOCK]
    lasts = batch[BLOCK - 1::BLOCK]
    b3 = batch.reshape(nb, 1, BLOCK).astype(jnp.int16)
    b1_2d = b1.reshape(1, D)
    W1 = W1.astype(jnp.bfloat16)
    W2 = W2.astype(jnp.bfloat16)

    grid_spec = pltpu.PrefetchScalarGridSpec(
        num_scalar_prefetch=2,
        grid=(nb,),
        in_specs=[
            pl.BlockSpec((1, 1, BLOCK), lambda i, *_: (i, 0, 0)),
            pl.BlockSpec((BLOCK, D), lambda i, *_: (i, 0)),
            pl.BlockSpec((D, D), lambda i, *_: (0, 0)),
            pl.BlockSpec((1, D), lambda i, *_: (0, 0)),
            pl.BlockSpec((1, D), lambda i, *_: (0, 0)),
        ],
        out_specs=pl.BlockSpec((NUM_GRAPHS, D), lambda i, *_: (0, 0)),
        scratch_shapes=[
            pltpu.VMEM((1, 1), jnp.float32),
            pltpu.VMEM((NUM_GRAPHS, 1), jnp.float32),
            pltpu.VMEM((NUM_GRAPHS, D), jnp.float32),
        ],
    )
    out = pl.pallas_call(
        functools.partial(_readout_kernel, nb=nb, n=n),
        grid_spec=grid_spec,
        out_shape=jax.ShapeDtypeStruct((NUM_GRAPHS, D), jnp.float32),
    )(firsts, lasts, b3, x, W1, b1_2d, W2)
    return out


# speedup vs baseline: 1.2416x; 1.2416x over previous
import functools

import jax
import jax.numpy as jnp
from jax.experimental import pallas as pl
from jax.experimental.pallas import tpu as pltpu

N_CONST = 100000
D = 128
NUM_GRAPHS = 512
BLOCK = 8192
SEG_TILE = 64
N_TILES = NUM_GRAPHS // SEG_TILE


def _readout_kernel(first_ref, last_ref, b_ref, x_ref, w1_ref, b1_ref,
                    w2_ref, out_ref, m_scr, s_scr, v_scr, *, nb, n):
    i = pl.program_id(0)

    @pl.when(i == 0)
    def _init():
        m_scr[...] = jnp.zeros_like(m_scr)
        s_scr[...] = jnp.zeros_like(s_scr)
        v_scr[...] = jnp.zeros_like(v_scr)

    x_blk = x_ref[...]
    if n % BLOCK:
        @pl.when(i == nb - 1)
        def _mask_tail():
            row = jax.lax.broadcasted_iota(jnp.int32, (BLOCK, 1), 0)
            x_ref[...] = jnp.where(row < (n - (nb - 1) * BLOCK),
                                   x_blk, 0.0)
        x_blk = x_ref[...]
    x_bf = x_blk.astype(jnp.bfloat16)
    h = jnp.tanh(
        jax.lax.dot_general(x_bf, w1_ref[...], (((1,), (1,)), ((), ())),
                            preferred_element_type=jnp.float32)
        + b1_ref[...])
    sc = jax.lax.dot_general(w2_ref[...], h.astype(jnp.bfloat16),
                             (((1,), (1,)), ((), ())),
                             preferred_element_type=jnp.float32)

    m_old = m_scr[...]
    m_new = jnp.maximum(m_old, jnp.max(sc, axis=(0, 1), keepdims=True))
    m_scr[...] = m_new
    e = jnp.exp(sc - m_new).astype(jnp.bfloat16)

    @pl.when(m_new[0, 0] > m_old[0, 0])
    def _rescale():
        r = jnp.exp(m_old - m_new)[0, 0]
        s_scr[...] = s_scr[...] * r
        v_scr[...] = v_scr[...] * r

    ids = b_ref[0]
    first, last = first_ref[i], last_ref[i]
    zero = jnp.zeros((), jnp.bfloat16)
    ones = jnp.ones((BLOCK, 1), jnp.bfloat16)

    wstart = jnp.minimum((first // 8) * 8, NUM_GRAPHS - SEG_TILE)
    wend = wstart + SEG_TILE
    w_iota = (wstart + jax.lax.broadcasted_iota(
        jnp.int32, (SEG_TILE, 1), 0)).astype(jnp.int16)
    ohw = jnp.where(w_iota == ids, e, zero)
    wsl = pl.ds(wstart, SEG_TILE)
    s_scr[wsl, :] += jnp.dot(ohw, ones,
                             preferred_element_type=jnp.float32)
    v_scr[wsl, :] += jnp.dot(ohw, x_bf,
                             preferred_element_type=jnp.float32)

    wend16 = wend.astype(jnp.int16)
    for j in range(N_TILES):
        lo = j * SEG_TILE

        @pl.when((last >= wend) & (lo + SEG_TILE - 1 >= wend) & (lo <= last))
        def _tile(lo=lo):
            seg_iota = lo + jax.lax.broadcasted_iota(
                jnp.int16, (SEG_TILE, 1), 0)
            ohw_fb = jnp.where((seg_iota == ids) & (ids >= wend16), e, zero)
            sl = pl.ds(lo, SEG_TILE)
            s_scr[sl, :] += jnp.dot(ohw_fb, ones,
                                    preferred_element_type=jnp.float32)
            v_scr[sl, :] += jnp.dot(ohw_fb, x_bf,
                                    preferred_element_type=jnp.float32)

    @pl.when(i == nb - 1)
    def _finish():
        s = s_scr[...]
        recip = jnp.where(s > 0.0, 1.0 / s, 0.0)
        out_ref[...] = v_scr[...] * recip


def kernel(x, batch, W1, b1, W2):
    n = x.shape[0]
    nb = (n + BLOCK - 1) // BLOCK
    npad = nb * BLOCK - n
    batch = batch.astype(jnp.int32)
    if npad:
        batch = jnp.pad(batch, (0, npad), constant_values=NUM_GRAPHS)
    firsts = batch[::BLOCK]
    lasts = batch[BLOCK - 1::BLOCK]
    b3 = batch.reshape(nb, 1, BLOCK).astype(jnp.int16)
    b1_2d = b1.reshape(1, D)
    W1 = W1.astype(jnp.bfloat16)
    W2 = W2.astype(jnp.bfloat16)

    grid_spec = pltpu.PrefetchScalarGridSpec(
        num_scalar_prefetch=2,
        grid=(nb,),
        in_specs=[
            pl.BlockSpec((1, 1, BLOCK), lambda i, *_: (i, 0, 0)),
            pl.BlockSpec((BLOCK, D), lambda i, *_: (i, 0)),
            pl.BlockSpec((D, D), lambda i, *_: (0, 0)),
            pl.BlockSpec((1, D), lambda i, *_: (0, 0)),
            pl.BlockSpec((1, D), lambda i, *_: (0, 0)),
        ],
        out_specs=pl.BlockSpec((NUM_GRAPHS, D), lambda i, *_: (0, 0)),
        scratch_shapes=[
            pltpu.VMEM((1, 1), jnp.float32),
            pltpu.VMEM((NUM_GRAPHS, 1), jnp.float32),
            pltpu.VMEM((NUM_GRAPHS, D), jnp.float32),
        ],
    )
    out = pl.pallas_call(
        functools.partial(_readout_kernel, nb=nb, n=n),
        grid_spec=grid_spec,
        out_shape=jax.ShapeDtypeStruct((NUM_GRAPHS, D), jnp.float32),
    )(firsts, lasts, b3, x, W1, b1_2d, W2)
    return out
